# baseline (speedup 1.0000x reference)
import jax
import jax.numpy as jnp
from jax.experimental import pallas as pl
from jax.experimental.pallas import tpu as pltpu

QK = 64
K_NN = 16
GEO = 12
SEM = 192
NC = 13


def _conv1d(x, W, b):
    return jnp.einsum('oc,bcn->bon', W, x) + b[None, :, None]


def _bn1d(x, g, be, eps=1e-5):
    m = jnp.mean(x, axis=(0, 2), keepdims=True)
    v = jnp.var(x, axis=(0, 2), keepdims=True)
    return (x - m) / jnp.sqrt(v + eps) * g[None, :, None] + be[None, :, None]


def _conv2d(x, W, b):
    return jnp.einsum('oc,bcnk->bonk', W, x) + b[None, :, None, None]


def _bn2d(x, g, be, eps=1e-5):
    m = jnp.mean(x, axis=(0, 2, 3), keepdims=True)
    v = jnp.var(x, axis=(0, 2, 3), keepdims=True)
    return (x - m) / jnp.sqrt(v + eps) * g[None, :, None, None] + be[None, :, None, None]


def _gather_val(tensor, idx):
    t = jnp.transpose(tensor, (0, 2, 1))
    val = jax.vmap(lambda tb, ib: tb[ib])(t, idx)
    return jnp.transpose(val, (0, 3, 1, 2))


def _knn_idx(xyz, k):
    B_, N_, _ = xyz.shape
    chunk = 2048
    outs = []
    for s in range(0, N_, chunk):
        q = xyz[:, s:s + chunk]
        d = jnp.sum((q[:, :, None, :] - xyz[:, None, :, :]) ** 2, axis=-1)
        _, idx = jax.lax.top_k(-d, k)
        outs.append(idx)
    return jnp.concatenate(outs, axis=1)


def _cls_kernel(rf_ref, w_ref, b_ref, out_ref):
    out_ref[...] = jnp.dot(rf_ref[...], w_ref[...],
                           preferred_element_type=jnp.float32) + b_ref[...]


def _cls_matmul(rf, cls_w, cls_b):
    M = rf.shape[0]
    wpad = jnp.zeros((QK, 128), jnp.float32).at[:, :NC].set(cls_w.T)
    bpad = jnp.zeros((1, 128), jnp.float32).at[0, :NC].set(cls_b)
    blk = 2048
    out = pl.pallas_call(
        _cls_kernel,
        grid=(M // blk,),
        in_specs=[pl.BlockSpec((blk, QK), lambda i: (i, 0)),
                  pl.BlockSpec((QK, 128), lambda i: (0, 0)),
                  pl.BlockSpec((1, 128), lambda i: (0, 0))],
        out_specs=pl.BlockSpec((blk, 128), lambda i: (i, 0)),
        out_shape=jax.ShapeDtypeStruct((M, 128), jnp.float32),
    )(rf, wpad, bpad)
    return out[:, :NC]


def kernel(xyz, jafar_feat, sem_feat, params):
    p = params
    k_idx = _knn_idx(xyz, K_NN)
    xyz_t = jnp.transpose(xyz, (0, 2, 1))
    jt = jnp.transpose(jafar_feat, (0, 2, 1))
    st = jnp.transpose(sem_feat, (0, 2, 1))
    g = jax.nn.relu(_bn1d(_conv1d(jt, p['ge_w1'], p['ge_b1']), p['ge_g1'], p['ge_be1']))
    geom_emb = jax.nn.relu(_bn1d(_conv1d(g, p['ge_w2'], p['ge_b2']), p['ge_g2'], p['ge_be2']))
    h = jax.nn.relu(_bn1d(_conv1d(geom_emb, p['bd_w1'], p['bd_b1']), p['bd_g'], p['bd_be']))
    bdy_logits = _conv1d(h, p['bd_w2'], p['bd_b2'])
    Q = _conv1d(geom_emb, p['q_w'], p['q_b'])
    Kt = _conv1d(geom_emb, p['k_w'], p['k_b'])
    V = jax.nn.relu(_bn1d(_conv1d(st, p['v_w'], p['v_b']), p['v_g'], p['v_be']))
    K_g = _gather_val(Kt, k_idx)
    xyz_g = _gather_val(xyz_t, k_idx)
    V_g = _gather_val(V, k_idx)
    rel_pos = xyz_t[..., None] - xyz_g
    pe = _conv2d(rel_pos, p['rp_w1'], p['rp_b1'])
    pe = jax.nn.relu(_bn2d(pe, p['rp_g'], p['rp_be']))
    pos_enc = _conv2d(pe, p['rp_w2'], p['rp_b2'])
    attn_logits = jnp.sum(Q[..., None] * (K_g + pos_enc), axis=1) / (QK ** 0.5)
    affinity = jax.nn.softmax(attn_logits, axis=-1)
    refined = jnp.sum(affinity[:, None, :, :] * V_g, axis=-1) + V
    rf = jnp.transpose(refined, (0, 2, 1)).reshape(-1, QK)
    logits = _cls_matmul(rf, p['cls_w'], p['cls_b'])
    return (logits, affinity, k_idx, rf, bdy_logits)


# speedup vs baseline: 11.3037x; 11.3037x over previous
import jax
import jax.numpy as jnp
import numpy as np
from jax.experimental import pallas as pl
from jax.experimental.pallas import tpu as pltpu

QK = 64
K_NN = 16
GEO = 12
SEM = 192
NC = 13
B = 2
N = 16384
NT = B * N
NK = NT * K_NN
EPS = 1e-5


_QB = 32
_BIG = np.int32(1 << 30)


def _knn_kernel(xq_ref, xc_ref, out_ref):
    qx = xq_ref[0, :, 0:1]
    qy = xq_ref[0, :, 1:2]
    qz = xq_ref[0, :, 2:3]
    cx = xc_ref[0, 0:1, :]
    cy = xc_ref[0, 1:2, :]
    cz = xc_ref[0, 2:3, :]
    dx = qx - cx
    dy = qy - cy
    dz = qz - cz
    d = (dx * dx + dy * dy) + dz * dz
    lane = jax.lax.broadcasted_iota(jnp.int32, (_QB, N), 1)
    cols = []
    for _ in range(K_NN):
        gmin = jnp.min(d, axis=1, keepdims=True)
        idx = jnp.min(jnp.where(d == gmin, lane, _BIG), axis=1)
        cols.append(idx[:, None])
        d = jnp.where(lane == idx[:, None], jnp.inf, d)
    out_ref[0] = jnp.concatenate(cols, axis=1)


def _knn(xyz4, xyzT):
    return pl.pallas_call(
        _knn_kernel,
        grid=(B, N // _QB),
        in_specs=[pl.BlockSpec((1, _QB, 4), lambda b, i: (b, i, 0)),
                  pl.BlockSpec((1, 3, N), lambda b, i: (b, 0, 0))],
        out_specs=pl.BlockSpec((1, _QB, K_NN), lambda b, i: (b, i, 0)),
        out_shape=jax.ShapeDtypeStruct((B, N, K_NN), jnp.int32),
    )(xyz4, xyzT)


_RB = 2048


def _mm_stats_kernel(x_ref, w_ref, b_ref, y_ref, st_ref):
    y = jnp.dot(x_ref[...], w_ref[...], preferred_element_type=jnp.float32)
    y = y + b_ref[...]
    y_ref[...] = y

    @pl.when(pl.program_id(0) == 0)
    def _():
        st_ref[...] = jnp.zeros_like(st_ref)

    st_ref[0:1, :] += jnp.sum(y, axis=0, keepdims=True)
    st_ref[1:2, :] += jnp.sum(y * y, axis=0, keepdims=True)


def _mm_stats(x, w, b):
    m, cin = x.shape
    cout = w.shape[1]
    return pl.pallas_call(
        _mm_stats_kernel,
        grid=(m // _RB,),
        in_specs=[pl.BlockSpec((_RB, cin), lambda i: (i, 0)),
                  pl.BlockSpec((cin, cout), lambda i: (0, 0)),
                  pl.BlockSpec((1, cout), lambda i: (0, 0))],
        out_specs=[pl.BlockSpec((_RB, cout), lambda i: (i, 0)),
                   pl.BlockSpec((8, cout), lambda i: (0, 0))],
        out_shape=[jax.ShapeDtypeStruct((m, cout), jnp.float32),
                   jax.ShapeDtypeStruct((8, cout), jnp.float32)],
    )(x, w, b)


def _aff_mm_stats_kernel(x_ref, s_ref, t_ref, w_ref, b_ref, y_ref, st_ref):
    a = jax.nn.relu(x_ref[...] * s_ref[...] + t_ref[...])
    y = jnp.dot(a, w_ref[...], preferred_element_type=jnp.float32)
    y = y + b_ref[...]
    y_ref[...] = y

    @pl.when(pl.program_id(0) == 0)
    def _():
        st_ref[...] = jnp.zeros_like(st_ref)

    st_ref[0:1, :] += jnp.sum(y, axis=0, keepdims=True)
    st_ref[1:2, :] += jnp.sum(y * y, axis=0, keepdims=True)


def _aff_mm_stats(x, s, t, w, b):
    m, cin = x.shape
    cout = w.shape[1]
    return pl.pallas_call(
        _aff_mm_stats_kernel,
        grid=(m // _RB,),
        in_specs=[pl.BlockSpec((_RB, cin), lambda i: (i, 0)),
                  pl.BlockSpec((1, cin), lambda i: (0, 0)),
                  pl.BlockSpec((1, cin), lambda i: (0, 0)),
                  pl.BlockSpec((cin, cout), lambda i: (0, 0)),
                  pl.BlockSpec((1, cout), lambda i: (0, 0))],
        out_specs=[pl.BlockSpec((_RB, cout), lambda i: (i, 0)),
                   pl.BlockSpec((8, cout), lambda i: (0, 0))],
        out_shape=[jax.ShapeDtypeStruct((m, cout), jnp.float32),
                   jax.ShapeDtypeStruct((8, cout), jnp.float32)],
    )(x, s, t, w, b)


def _qkh_kernel(x_ref, s_ref, t_ref, qw_ref, qb_ref, kw_ref, kb_ref,
                bw_ref, bb_ref, q_ref, k_ref, h_ref, st_ref):
    ge = jax.nn.relu(x_ref[...] * s_ref[...] + t_ref[...])
    q_ref[...] = jnp.dot(ge, qw_ref[...], preferred_element_type=jnp.float32) + qb_ref[...]
    k_ref[...] = jnp.dot(ge, kw_ref[...], preferred_element_type=jnp.float32) + kb_ref[...]
    h = jnp.dot(ge, bw_ref[...], preferred_element_type=jnp.float32) + bb_ref[...]
    h_ref[...] = h

    @pl.when(pl.program_id(0) == 0)
    def _():
        st_ref[...] = jnp.zeros_like(st_ref)

    st_ref[0:1, :] += jnp.sum(h, axis=0, keepdims=True)
    st_ref[1:2, :] += jnp.sum(h * h, axis=0, keepdims=True)


def _qkh(y2, s, t, qw, qb, kw, kb, bw, bb):
    return pl.pallas_call(
        _qkh_kernel,
        grid=(NT // _RB,),
        in_specs=[pl.BlockSpec((_RB, QK), lambda i: (i, 0)),
                  pl.BlockSpec((1, QK), lambda i: (0, 0)),
                  pl.BlockSpec((1, QK), lambda i: (0, 0)),
                  pl.BlockSpec((QK, QK), lambda i: (0, 0)),
                  pl.BlockSpec((1, QK), lambda i: (0, 0)),
                  pl.BlockSpec((QK, QK), lambda i: (0, 0)),
                  pl.BlockSpec((1, QK), lambda i: (0, 0)),
                  pl.BlockSpec((QK, 32), lambda i: (0, 0)),
                  pl.BlockSpec((1, 32), lambda i: (0, 0))],
        out_specs=[pl.BlockSpec((_RB, QK), lambda i: (i, 0)),
                   pl.BlockSpec((_RB, QK), lambda i: (i, 0)),
                   pl.BlockSpec((_RB, 32), lambda i: (i, 0)),
                   pl.BlockSpec((8, 32), lambda i: (0, 0))],
        out_shape=[jax.ShapeDtypeStruct((NT, QK), jnp.float32),
                   jax.ShapeDtypeStruct((NT, QK), jnp.float32),
                   jax.ShapeDtypeStruct((NT, 32), jnp.float32),
                   jax.ShapeDtypeStruct((8, 32), jnp.float32)],
    )(y2, s, t, qw, qb, kw, kb, bw, bb)


def _vbdy_kernel(vy_ref, sv_ref, tv_ref, h_ref, sh_ref, th_ref,
                 bw2_ref, bb2_ref, v_ref, bdy_ref):
    v_ref[...] = jax.nn.relu(vy_ref[...] * sv_ref[...] + tv_ref[...])
    hh = jax.nn.relu(h_ref[...] * sh_ref[...] + th_ref[...])
    bd = jnp.sum(hh * bw2_ref[...], axis=1)
    bdy_ref[...] = bd.reshape(_RB // 128, 128) + bb2_ref[...]


def _vbdy(vy, sv, tv, h, sh, th, bw2, bb2):
    return pl.pallas_call(
        _vbdy_kernel,
        grid=(NT // _RB,),
        in_specs=[pl.BlockSpec((_RB, QK), lambda i: (i, 0)),
                  pl.BlockSpec((1, QK), lambda i: (0, 0)),
                  pl.BlockSpec((1, QK), lambda i: (0, 0)),
                  pl.BlockSpec((_RB, 32), lambda i: (i, 0)),
                  pl.BlockSpec((1, 32), lambda i: (0, 0)),
                  pl.BlockSpec((1, 32), lambda i: (0, 0)),
                  pl.BlockSpec((1, 32), lambda i: (0, 0)),
                  pl.BlockSpec((1, 128), lambda i: (0, 0))],
        out_specs=[pl.BlockSpec((_RB, QK), lambda i: (i, 0)),
                   pl.BlockSpec((_RB // 128, 128), lambda i: (i, 0))],
        out_shape=[jax.ShapeDtypeStruct((NT, QK), jnp.float32),
                   jax.ShapeDtypeStruct((NT // 128, 128), jnp.float32)],
    )(vy, sv, tv, h, sh, th, bw2, bb2)


_MB = 8192


def _mom_kernel(rp_ref, m_ref):
    rp = rp_ref[...]
    acc = jax.lax.dot_general(rp, rp, (((0,), (0,)), ((), ())),
                              preferred_element_type=jnp.float32)

    @pl.when(pl.program_id(0) == 0)
    def _():
        m_ref[...] = jnp.zeros_like(m_ref)

    m_ref[...] += acc


def _moments(rp):
    return pl.pallas_call(
        _mom_kernel,
        grid=(NK // _MB,),
        in_specs=[pl.BlockSpec((_MB, 16), lambda i: (i, 0))],
        out_specs=pl.BlockSpec((16, 16), lambda i: (0, 0)),
        out_shape=jax.ShapeDtypeStruct((16, 16), jnp.float32),
    )(rp)


_AB = 512


def _attn_kernel(q_ref, v_ref, kg_ref, vg_ref, rp_ref, wp1_ref, sp_ref,
                 tp_ref, w2_ref, b2_ref, cw_ref, cb_ref,
                 logit_ref, aff_ref, rf_ref):
    rp = rp_ref[...]
    pe1 = jnp.dot(rp, wp1_ref[...], preferred_element_type=jnp.float32)
    a1 = jax.nn.relu(pe1 * sp_ref[...] + tp_ref[...])
    pos = jnp.dot(a1, w2_ref[...], preferred_element_type=jnp.float32) + b2_ref[...]
    kgp = (kg_ref[...] + pos).reshape(_AB, K_NN, QK)
    q = q_ref[...]
    attn = jnp.sum(kgp * q[:, None, :], axis=2) * (1.0 / (QK ** 0.5))
    amax = jnp.max(attn, axis=1, keepdims=True)
    ex = jnp.exp(attn - amax)
    aff = ex / jnp.sum(ex, axis=1, keepdims=True)
    aff_ref[...] = aff
    vg = vg_ref[...].reshape(_AB, K_NN, QK)
    refined = jnp.sum(aff[:, :, None] * vg, axis=1) + v_ref[...]
    rf_ref[...] = refined
    logit_ref[...] = jnp.dot(refined, cw_ref[...],
                             preferred_element_type=jnp.float32) + cb_ref[...]


def _attention(q, v, kg, vg, rp, wp1, sp, tp, w2, b2, cw, cb):
    g = K_NN * _AB
    return pl.pallas_call(
        _attn_kernel,
        grid=(NT // _AB,),
        in_specs=[pl.BlockSpec((_AB, QK), lambda i: (i, 0)),
                  pl.BlockSpec((_AB, QK), lambda i: (i, 0)),
                  pl.BlockSpec((g, QK), lambda i: (i, 0)),
                  pl.BlockSpec((g, QK), lambda i: (i, 0)),
                  pl.BlockSpec((g, 16), lambda i: (i, 0)),
                  pl.BlockSpec((16, QK), lambda i: (0, 0)),
                  pl.BlockSpec((1, QK), lambda i: (0, 0)),
                  pl.BlockSpec((1, QK), lambda i: (0, 0)),
                  pl.BlockSpec((QK, QK), lambda i: (0, 0)),
                  pl.BlockSpec((1, QK), lambda i: (0, 0)),
                  pl.BlockSpec((QK, 128), lambda i: (0, 0)),
                  pl.BlockSpec((1, 128), lambda i: (0, 0))],
        out_specs=[pl.BlockSpec((_AB, 128), lambda i: (i, 0)),
                   pl.BlockSpec((_AB, K_NN), lambda i: (i, 0)),
                   pl.BlockSpec((_AB, QK), lambda i: (i, 0))],
        out_shape=[jax.ShapeDtypeStruct((NT, 128), jnp.float32),
                   jax.ShapeDtypeStruct((NT, K_NN), jnp.float32),
                   jax.ShapeDtypeStruct((NT, QK), jnp.float32)],
    )(q, v, kg, vg, rp, wp1, sp, tp, w2, b2, cw, cb)


def _bn_finalize(st, g, be, cnt):
    mean = st[0] / cnt
    var = st[1] / cnt - mean * mean
    s = g / jnp.sqrt(var + EPS)
    t = be - mean * s
    return s[None, :], t[None, :]


def kernel(xyz, jafar_feat, sem_feat, params):
    p = params

    xyz4 = jnp.concatenate(
        [xyz, jnp.zeros((B, N, 1), jnp.float32)], axis=2)
    xyzT = jnp.transpose(xyz, (0, 2, 1))
    k_idx = _knn(xyz4, xyzT)

    jaf2 = jafar_feat.reshape(NT, GEO)
    sem2 = sem_feat.reshape(NT, SEM)

    y1, st1 = _mm_stats(jaf2, p['ge_w1'].T, p['ge_b1'][None, :])
    s1, t1 = _bn_finalize(st1, p['ge_g1'], p['ge_be1'], NT)
    y2, st2 = _aff_mm_stats(y1, s1, t1, p['ge_w2'].T, p['ge_b2'][None, :])
    s2, t2 = _bn_finalize(st2, p['ge_g2'], p['ge_be2'], NT)
    q_arr, kt, h, sth = _qkh(y2, s2, t2, p['q_w'].T, p['q_b'][None, :],
                             p['k_w'].T, p['k_b'][None, :],
                             p['bd_w1'].T, p['bd_b1'][None, :])
    shh, thh = _bn_finalize(sth, p['bd_g'], p['bd_be'], NT)
    vy, stv = _mm_stats(sem2, p['v_w'].T, p['v_b'][None, :])
    sv, tv = _bn_finalize(stv, p['v_g'], p['v_be'], NT)
    v_arr, bdyr = _vbdy(vy, sv, tv, h, shh, thh,
                        p['bd_w2'][0][None, :],
                        jnp.full((1, 128), p['bd_b2'][0]))
    bdy_logits = bdyr.reshape(B, 1, N)

    gidx = (k_idx + jnp.arange(B, dtype=jnp.int32)[:, None, None] * N
            ).reshape(NK)
    kg = kt[gidx]
    vg = v_arr[gidx]
    xyzf = xyz.reshape(NT, 3)
    xg = xyzf[gidx]
    rp3 = jnp.repeat(xyzf, K_NN, axis=0) - xg
    rp = jnp.concatenate(
        [rp3, jnp.ones((NK, 1), jnp.float32),
         jnp.zeros((NK, 12), jnp.float32)], axis=1)

    m = _moments(rp)
    cnt = jnp.float32(NK)
    mu = m[3, :3] / cnt
    srr = m[:3, :3] / cnt
    w1 = p['rp_w1']
    b1 = p['rp_b1']
    mean_pe = w1 @ mu + b1
    e2_pe = jnp.einsum('oc,cd,od->o', w1, srr, w1) + 2.0 * b1 * (w1 @ mu) + b1 * b1
    var_pe = e2_pe - mean_pe * mean_pe
    sp = (p['rp_g'] / jnp.sqrt(var_pe + EPS))[None, :]
    tp = (p['rp_be'] - mean_pe * sp[0])[None, :]

    wp1 = jnp.zeros((16, QK), jnp.float32)
    wp1 = wp1.at[:3, :].set(w1.T).at[3, :].set(b1)

    cw = jnp.zeros((QK, 128), jnp.float32).at[:, :NC].set(p['cls_w'].T)
    cb = jnp.zeros((1, 128), jnp.float32).at[0, :NC].set(p['cls_b'])

    logitsP, aff, rf = _attention(q_arr, v_arr, kg, vg, rp, wp1, sp, tp,
                                  p['rp_w2'].T, p['rp_b2'][None, :], cw, cb)

    logits = logitsP[:, :NC]
    affinity = aff.reshape(B, N, K_NN)
    return (logits, affinity, k_idx, rf, bdy_logits)


# speedup vs baseline: 15.2060x; 1.3452x over previous
import jax
import jax.numpy as jnp
import numpy as np
from jax import lax
from jax.experimental import pallas as pl
from jax.experimental.pallas import tpu as pltpu
from jax.experimental.pallas import tpu_sc as plsc

QK = 64
K_NN = 16
GEO = 12
SEM = 192
NC = 13
B = 2
N = 16384
NT = B * N
NK = NT * K_NN
EPS = 1e-5


_QB = 32
_BIG = np.int32(1 << 30)


def _knn_kernel(xq_ref, xc_ref, out_ref):
    qx = xq_ref[0, :, 0:1]
    qy = xq_ref[0, :, 1:2]
    qz = xq_ref[0, :, 2:3]
    cx = xc_ref[0, 0:1, :]
    cy = xc_ref[0, 1:2, :]
    cz = xc_ref[0, 2:3, :]
    dx = qx - cx
    dy = qy - cy
    dz = qz - cz
    d = (dx * dx + dy * dy) + dz * dz
    lane = jax.lax.broadcasted_iota(jnp.int32, (_QB, N), 1)
    cols = []
    for _ in range(K_NN):
        gmin = jnp.min(d, axis=1, keepdims=True)
        idx = jnp.min(jnp.where(d == gmin, lane, _BIG), axis=1)
        cols.append(idx[:, None])
        d = jnp.where(lane == idx[:, None], jnp.inf, d)
    out_ref[0] = jnp.concatenate(cols, axis=1)


def _knn(xyz4, xyzT):
    return pl.pallas_call(
        _knn_kernel,
        grid=(B, N // _QB),
        in_specs=[pl.BlockSpec((1, _QB, 4), lambda b, i: (b, i, 0)),
                  pl.BlockSpec((1, 3, N), lambda b, i: (b, 0, 0))],
        out_specs=pl.BlockSpec((1, _QB, K_NN), lambda b, i: (b, i, 0)),
        out_shape=jax.ShapeDtypeStruct((B, N, K_NN), jnp.int32),
    )(xyz4, xyzT)


_NW = 32
_GC = 128


def _sc_gather_body(gidx_hbm, kt_hbm, v_hbm, xyzp_hbm,
                    kg_hbm, vg_hbm, rp_hbm,
                    idx_v, rows_v, rowsx_v, qx_v, sem):
    wid = lax.axis_index("s") * 2 + lax.axis_index("c")
    per_w = NK // _NW
    base = wid * per_w
    ones3 = jnp.where(lax.iota(jnp.int32, 16) == 3,
                      jnp.float32(1.0), jnp.float32(0.0))

    def chunk(i, _):
        off = pl.multiple_of(base + i * _GC, _GC)
        qoff = pl.multiple_of(off // K_NN, _GC // K_NN)
        pltpu.sync_copy(gidx_hbm.at[pl.ds(off, _GC)], idx_v)
        pltpu.async_copy(kt_hbm.at[idx_v], rows_v, sem).wait()
        pltpu.sync_copy(rows_v, kg_hbm.at[pl.ds(off, _GC)])
        pltpu.async_copy(v_hbm.at[idx_v], rows_v, sem).wait()
        pltpu.sync_copy(rows_v, vg_hbm.at[pl.ds(off, _GC)])
        pltpu.async_copy(xyzp_hbm.at[idx_v], rowsx_v, sem).wait()
        pltpu.sync_copy(xyzp_hbm.at[pl.ds(qoff, _GC // K_NN)], qx_v)
        for j in range(_GC // K_NN):
            qp = qx_v[j, pl.ds(0, 16)] + ones3
            for t in range(K_NN):
                r = j * K_NN + t
                rowsx_v[r, pl.ds(0, 16)] = qp - rowsx_v[r, pl.ds(0, 16)]
        pltpu.sync_copy(rowsx_v, rp_hbm.at[pl.ds(off, _GC)])
        return 0

    lax.fori_loop(0, per_w // _GC, chunk, 0)


def _sc_gather(gidx, kt, v, xyzp):
    mesh = plsc.VectorSubcoreMesh(core_axis_name="c", subcore_axis_name="s")
    f = pl.kernel(
        _sc_gather_body,
        out_type=[jax.ShapeDtypeStruct((NK, 128), jnp.float32),
                  jax.ShapeDtypeStruct((NK, 128), jnp.float32),
                  jax.ShapeDtypeStruct((NK, 128), jnp.float32)],
        mesh=mesh,
        scratch_types=[pltpu.VMEM((_GC,), jnp.int32),
                       pltpu.VMEM((_GC, 128), jnp.float32),
                       pltpu.VMEM((_GC, 128), jnp.float32),
                       pltpu.VMEM((_GC // K_NN, 128), jnp.float32),
                       pltpu.SemaphoreType.DMA],
    )
    return f(gidx, kt, v, xyzp)


_RB = 2048


def _mm_stats_kernel(x_ref, w_ref, b_ref, y_ref, st_ref):
    y = jnp.dot(x_ref[...], w_ref[...], preferred_element_type=jnp.float32)
    y = y + b_ref[...]
    y_ref[...] = y

    @pl.when(pl.program_id(0) == 0)
    def _():
        st_ref[...] = jnp.zeros_like(st_ref)

    st_ref[0:1, :] += jnp.sum(y, axis=0, keepdims=True)
    st_ref[1:2, :] += jnp.sum(y * y, axis=0, keepdims=True)


def _mm_stats(x, w, b):
    m, cin = x.shape
    cout = w.shape[1]
    return pl.pallas_call(
        _mm_stats_kernel,
        grid=(m // _RB,),
        in_specs=[pl.BlockSpec((_RB, cin), lambda i: (i, 0)),
                  pl.BlockSpec((cin, cout), lambda i: (0, 0)),
                  pl.BlockSpec((1, cout), lambda i: (0, 0))],
        out_specs=[pl.BlockSpec((_RB, cout), lambda i: (i, 0)),
                   pl.BlockSpec((8, cout), lambda i: (0, 0))],
        out_shape=[jax.ShapeDtypeStruct((m, cout), jnp.float32),
                   jax.ShapeDtypeStruct((8, cout), jnp.float32)],
    )(x, w, b)


def _aff_mm_stats_kernel(x_ref, s_ref, t_ref, w_ref, b_ref, y_ref, st_ref):
    a = jax.nn.relu(x_ref[...] * s_ref[...] + t_ref[...])
    y = jnp.dot(a, w_ref[...], preferred_element_type=jnp.float32)
    y = y + b_ref[...]
    y_ref[...] = y

    @pl.when(pl.program_id(0) == 0)
    def _():
        st_ref[...] = jnp.zeros_like(st_ref)

    st_ref[0:1, :] += jnp.sum(y, axis=0, keepdims=True)
    st_ref[1:2, :] += jnp.sum(y * y, axis=0, keepdims=True)


def _aff_mm_stats(x, s, t, w, b):
    m, cin = x.shape
    cout = w.shape[1]
    return pl.pallas_call(
        _aff_mm_stats_kernel,
        grid=(m // _RB,),
        in_specs=[pl.BlockSpec((_RB, cin), lambda i: (i, 0)),
                  pl.BlockSpec((1, cin), lambda i: (0, 0)),
                  pl.BlockSpec((1, cin), lambda i: (0, 0)),
                  pl.BlockSpec((cin, cout), lambda i: (0, 0)),
                  pl.BlockSpec((1, cout), lambda i: (0, 0))],
        out_specs=[pl.BlockSpec((_RB, cout), lambda i: (i, 0)),
                   pl.BlockSpec((8, cout), lambda i: (0, 0))],
        out_shape=[jax.ShapeDtypeStruct((m, cout), jnp.float32),
                   jax.ShapeDtypeStruct((8, cout), jnp.float32)],
    )(x, s, t, w, b)


def _qkh_kernel(x_ref, s_ref, t_ref, qw_ref, qb_ref, kw_ref, kb_ref,
                bw_ref, bb_ref, q_ref, k_ref, h_ref, st_ref):
    ge = jax.nn.relu(x_ref[...] * s_ref[...] + t_ref[...])
    q_ref[...] = jnp.dot(ge, qw_ref[...], preferred_element_type=jnp.float32) + qb_ref[...]
    k_ref[...] = jnp.dot(ge, kw_ref[...], preferred_element_type=jnp.float32) + kb_ref[...]
    h = jnp.dot(ge, bw_ref[...], preferred_element_type=jnp.float32) + bb_ref[...]
    h_ref[...] = h

    @pl.when(pl.program_id(0) == 0)
    def _():
        st_ref[...] = jnp.zeros_like(st_ref)

    st_ref[0:1, :] += jnp.sum(h, axis=0, keepdims=True)
    st_ref[1:2, :] += jnp.sum(h * h, axis=0, keepdims=True)


def _qkh(y2, s, t, qw, qb, kw, kb, bw, bb):
    return pl.pallas_call(
        _qkh_kernel,
        grid=(NT // _RB,),
        in_specs=[pl.BlockSpec((_RB, QK), lambda i: (i, 0)),
                  pl.BlockSpec((1, QK), lambda i: (0, 0)),
                  pl.BlockSpec((1, QK), lambda i: (0, 0)),
                  pl.BlockSpec((QK, QK), lambda i: (0, 0)),
                  pl.BlockSpec((1, QK), lambda i: (0, 0)),
                  pl.BlockSpec((QK, 128), lambda i: (0, 0)),
                  pl.BlockSpec((1, 128), lambda i: (0, 0)),
                  pl.BlockSpec((QK, 32), lambda i: (0, 0)),
                  pl.BlockSpec((1, 32), lambda i: (0, 0))],
        out_specs=[pl.BlockSpec((_RB, QK), lambda i: (i, 0)),
                   pl.BlockSpec((_RB, 128), lambda i: (i, 0)),
                   pl.BlockSpec((_RB, 32), lambda i: (i, 0)),
                   pl.BlockSpec((8, 32), lambda i: (0, 0))],
        out_shape=[jax.ShapeDtypeStruct((NT, QK), jnp.float32),
                   jax.ShapeDtypeStruct((NT, 128), jnp.float32),
                   jax.ShapeDtypeStruct((NT, 32), jnp.float32),
                   jax.ShapeDtypeStruct((8, 32), jnp.float32)],
    )(y2, s, t, qw, qb, kw, kb, bw, bb)


def _vbdy_kernel(vy_ref, sv_ref, tv_ref, h_ref, sh_ref, th_ref,
                 bw2_ref, bb2_ref, v_ref, bdy_ref):
    v = jax.nn.relu(vy_ref[...] * sv_ref[...] + tv_ref[...])
    v_ref[...] = jnp.concatenate(
        [v, jnp.zeros((_RB, 128 - QK), jnp.float32)], axis=1)
    hh = jax.nn.relu(h_ref[...] * sh_ref[...] + th_ref[...])
    bd = jnp.sum(hh * bw2_ref[...], axis=1)
    bdy_ref[...] = bd.reshape(_RB // 128, 128) + bb2_ref[...]


def _vbdy(vy, sv, tv, h, sh, th, bw2, bb2):
    return pl.pallas_call(
        _vbdy_kernel,
        grid=(NT // _RB,),
        in_specs=[pl.BlockSpec((_RB, QK), lambda i: (i, 0)),
                  pl.BlockSpec((1, QK), lambda i: (0, 0)),
                  pl.BlockSpec((1, QK), lambda i: (0, 0)),
                  pl.BlockSpec((_RB, 32), lambda i: (i, 0)),
                  pl.BlockSpec((1, 32), lambda i: (0, 0)),
                  pl.BlockSpec((1, 32), lambda i: (0, 0)),
                  pl.BlockSpec((1, 32), lambda i: (0, 0)),
                  pl.BlockSpec((1, 128), lambda i: (0, 0))],
        out_specs=[pl.BlockSpec((_RB, 128), lambda i: (i, 0)),
                   pl.BlockSpec((_RB // 128, 128), lambda i: (i, 0))],
        out_shape=[jax.ShapeDtypeStruct((NT, 128), jnp.float32),
                   jax.ShapeDtypeStruct((NT // 128, 128), jnp.float32)],
    )(vy, sv, tv, h, sh, th, bw2, bb2)


_MB = 8192


def _mom_kernel(rp_ref, m_ref):
    rp = rp_ref[:, 0:16]
    acc = jax.lax.dot_general(rp, rp, (((0,), (0,)), ((), ())),
                              preferred_element_type=jnp.float32)

    @pl.when(pl.program_id(0) == 0)
    def _():
        m_ref[...] = jnp.zeros_like(m_ref)

    m_ref[...] += acc


def _moments(rp):
    return pl.pallas_call(
        _mom_kernel,
        grid=(NK // _MB,),
        in_specs=[pl.BlockSpec((_MB, 128), lambda i: (i, 0))],
        out_specs=pl.BlockSpec((16, 16), lambda i: (0, 0)),
        out_shape=jax.ShapeDtypeStruct((16, 16), jnp.float32),
    )(rp)


_AB = 512


def _attn_kernel(q_ref, v_ref, kg_ref, vg_ref, rp_ref, wp1_ref, sp_ref,
                 tp_ref, w2_ref, b2_ref, cw_ref, cb_ref,
                 logit_ref, aff_ref, rf_ref):
    rp = rp_ref[:, 0:16]
    pe1 = jnp.dot(rp, wp1_ref[...], preferred_element_type=jnp.float32)
    a1 = jax.nn.relu(pe1 * sp_ref[...] + tp_ref[...])
    pos = jnp.dot(a1, w2_ref[...], preferred_element_type=jnp.float32) + b2_ref[...]
    kgp = (kg_ref[:, 0:QK] + pos).reshape(_AB, K_NN, QK)
    q = q_ref[...]
    attn = jnp.sum(kgp * q[:, None, :], axis=2) * (1.0 / (QK ** 0.5))
    amax = jnp.max(attn, axis=1, keepdims=True)
    ex = jnp.exp(attn - amax)
    aff = ex / jnp.sum(ex, axis=1, keepdims=True)
    aff_ref[...] = aff
    vg = vg_ref[:, 0:QK].reshape(_AB, K_NN, QK)
    refined = jnp.sum(aff[:, :, None] * vg, axis=1) + v_ref[:, 0:QK]
    rf_ref[...] = refined
    logit_ref[...] = jnp.dot(refined, cw_ref[...],
                             preferred_element_type=jnp.float32) + cb_ref[...]


def _attention(q, v, kg, vg, rp, wp1, sp, tp, w2, b2, cw, cb):
    g = K_NN * _AB
    return pl.pallas_call(
        _attn_kernel,
        grid=(NT // _AB,),
        in_specs=[pl.BlockSpec((_AB, QK), lambda i: (i, 0)),
                  pl.BlockSpec((_AB, 128), lambda i: (i, 0)),
                  pl.BlockSpec((g, 128), lambda i: (i, 0)),
                  pl.BlockSpec((g, 128), lambda i: (i, 0)),
                  pl.BlockSpec((g, 128), lambda i: (i, 0)),
                  pl.BlockSpec((16, QK), lambda i: (0, 0)),
                  pl.BlockSpec((1, QK), lambda i: (0, 0)),
                  pl.BlockSpec((1, QK), lambda i: (0, 0)),
                  pl.BlockSpec((QK, QK), lambda i: (0, 0)),
                  pl.BlockSpec((1, QK), lambda i: (0, 0)),
                  pl.BlockSpec((QK, 128), lambda i: (0, 0)),
                  pl.BlockSpec((1, 128), lambda i: (0, 0))],
        out_specs=[pl.BlockSpec((_AB, 128), lambda i: (i, 0)),
                   pl.BlockSpec((_AB, K_NN), lambda i: (i, 0)),
                   pl.BlockSpec((_AB, QK), lambda i: (i, 0))],
        out_shape=[jax.ShapeDtypeStruct((NT, 128), jnp.float32),
                   jax.ShapeDtypeStruct((NT, K_NN), jnp.float32),
                   jax.ShapeDtypeStruct((NT, QK), jnp.float32)],
    )(q, v, kg, vg, rp, wp1, sp, tp, w2, b2, cw, cb)


def _bn_finalize(st, g, be, cnt):
    mean = st[0] / cnt
    var = st[1] / cnt - mean * mean
    s = g / jnp.sqrt(var + EPS)
    t = be - mean * s
    return s[None, :], t[None, :]


def kernel(xyz, jafar_feat, sem_feat, params):
    p = params

    xyz4 = jnp.concatenate(
        [xyz, jnp.zeros((B, N, 1), jnp.float32)], axis=2)
    xyzT = jnp.transpose(xyz, (0, 2, 1))
    k_idx = _knn(xyz4, xyzT)

    jaf2 = jafar_feat.reshape(NT, GEO)
    sem2 = sem_feat.reshape(NT, SEM)

    y1, st1 = _mm_stats(jaf2, p['ge_w1'].T, p['ge_b1'][None, :])
    s1, t1 = _bn_finalize(st1, p['ge_g1'], p['ge_be1'], NT)
    y2, st2 = _aff_mm_stats(y1, s1, t1, p['ge_w2'].T, p['ge_b2'][None, :])
    s2, t2 = _bn_finalize(st2, p['ge_g2'], p['ge_be2'], NT)
    kwp = jnp.zeros((QK, 128), jnp.float32).at[:, :QK].set(p['k_w'].T)
    kbp = jnp.zeros((1, 128), jnp.float32).at[0, :QK].set(p['k_b'])
    q_arr, kt, h, sth = _qkh(y2, s2, t2, p['q_w'].T, p['q_b'][None, :],
                             kwp, kbp,
                             p['bd_w1'].T, p['bd_b1'][None, :])
    shh, thh = _bn_finalize(sth, p['bd_g'], p['bd_be'], NT)
    vy, stv = _mm_stats(sem2, p['v_w'].T, p['v_b'][None, :])
    sv, tv = _bn_finalize(stv, p['v_g'], p['v_be'], NT)
    v_arr, bdyr = _vbdy(vy, sv, tv, h, shh, thh,
                        p['bd_w2'][0][None, :],
                        jnp.full((1, 128), p['bd_b2'][0]))
    bdy_logits = bdyr.reshape(B, 1, N)

    gidx = (k_idx + jnp.arange(B, dtype=jnp.int32)[:, None, None] * N
            ).reshape(NK)
    xyzp = jnp.concatenate(
        [xyz.reshape(NT, 3), jnp.zeros((NT, 125), jnp.float32)], axis=1)
    kg, vg, rp = _sc_gather(gidx, kt, v_arr, xyzp)

    m = _moments(rp)
    cnt = jnp.float32(NK)
    mu = m[3, :3] / cnt
    srr = m[:3, :3] / cnt
    w1 = p['rp_w1']
    b1 = p['rp_b1']
    mean_pe = w1 @ mu + b1
    e2_pe = jnp.einsum('oc,cd,od->o', w1, srr, w1) + 2.0 * b1 * (w1 @ mu) + b1 * b1
    var_pe = e2_pe - mean_pe * mean_pe
    sp = (p['rp_g'] / jnp.sqrt(var_pe + EPS))[None, :]
    tp = (p['rp_be'] - mean_pe * sp[0])[None, :]

    wp1 = jnp.zeros((16, QK), jnp.float32)
    wp1 = wp1.at[:3, :].set(w1.T).at[3, :].set(b1)

    cw = jnp.zeros((QK, 128), jnp.float32).at[:, :NC].set(p['cls_w'].T)
    cb = jnp.zeros((1, 128), jnp.float32).at[0, :NC].set(p['cls_b'])

    logitsP, aff, rf = _attention(q_arr, v_arr, kg, vg, rp, wp1, sp, tp,
                                  p['rp_w2'].T, p['rp_b2'][None, :], cw, cb)

    logits = logitsP[:, :NC]
    affinity = aff.reshape(B, N, K_NN)
    return (logits, affinity, k_idx, rf, bdy_logits)
